# baseline (speedup 1.0000x reference)
import functools

import numpy as np
import jax
import jax.numpy as jnp
from jax import lax
from jax.experimental import pallas as pl
from jax.experimental.pallas import tpu as pltpu
from jax.experimental.pallas import tpu_sc as plsc

N = 10000
D = 128
E = 320000
NC = 2
NS = 16
NW = NC * NS
EP = 327680
ET = EP // NW
K = 64
NCHUNK = ET // K
NBUF = 4
KS = 512
NCHS = ET // KS
N2 = 10240
STRIPE = N2 // NS
EXP1 = float(np.exp(np.float32(1.0)))

_MESH = dict(core_axis_name="c", subcore_axis_name="s")


def _iota16():
    return lax.iota(jnp.int32, 16)


@functools.partial(
    pl.kernel,
    mesh=plsc.VectorSubcoreMesh(**_MESH),
    compiler_params=pltpu.CompilerParams(needs_layout_passes=False),
    out_type=(
        jax.ShapeDtypeStruct((NC, N2), jnp.float32),
        jax.ShapeDtypeStruct((NC, N2), jnp.float32),
    ),
    scratch_types=[
        [pltpu.VMEM((KS * 2,), jnp.int32) for _ in range(3)],
        pltpu.VMEM((KS,), jnp.int32),
        pltpu.VMEM((KS,), jnp.float32),
        pltpu.VMEM((KS,), jnp.float32),
        pltpu.VMEM((STRIPE,), jnp.float32),
        pltpu.VMEM_SHARED((N2,), jnp.float32),
        pltpu.VMEM_SHARED((N2,), jnp.float32),
        [pltpu.SemaphoreType.DMA for _ in range(3)],
    ],
)
def _scalar_pass(ed2_hbm, d_out, c_out,
                 eb, dstb, valsb, onesb, zv, d_sh, c_sh, sem_i):
    c = lax.axis_index("c")
    s = lax.axis_index("s")

    def zbody(j, _):
        zv[pl.ds(j * 16, 16)] = jnp.zeros((16,), jnp.float32)
        return 0
    lax.fori_loop(0, STRIPE // 16, zbody, 0)
    pltpu.sync_copy(zv, d_sh.at[pl.ds(s * STRIPE, STRIPE)])
    pltpu.sync_copy(zv, c_sh.at[pl.ds(s * STRIPE, STRIPE)])
    plsc.subcore_barrier()

    base0 = (c * (EP // NC) + s * ET) * 2

    def istart(i, b):
        pltpu.async_copy(ed2_hbm.at[pl.ds(base0 + i * KS * 2, KS * 2)],
                         eb[b], sem_i[b])

    def iwait(i, b):
        pltpu.make_async_copy(ed2_hbm.at[pl.ds(base0 + i * KS * 2, KS * 2)],
                              eb[b], sem_i[b]).wait()

    istart(0, 0)
    istart(1, 1)
    iot2 = _iota16() * 2
    for i in range(NCHS):
        b = i % 3
        iwait(i, b)

        def compute(g, _):
            sl = pl.ds(g * 16, 16)
            idx = iot2 + g * 32
            dstb[sl] = plsc.load_gather(eb[b], [idx])
            w16 = plsc.bitcast(plsc.load_gather(eb[b], [idx + 1]),
                               jnp.float32)
            valsb[sl] = jnp.exp(w16)
            onesb[sl] = jnp.where(w16 > jnp.float32(-50.0),
                                  jnp.float32(1.0), jnp.float32(0.0))
            return 0
        lax.fori_loop(0, KS // 16, compute, 0, unroll=2)
        pltpu.sync_copy(valsb, d_sh.at[dstb], add=True)
        pltpu.sync_copy(onesb, c_sh.at[dstb], add=True)
        if i + 2 < NCHS:
            istart(i + 2, (i + 2) % 3)
    plsc.subcore_barrier()
    pltpu.sync_copy(d_sh.at[pl.ds(s * STRIPE, STRIPE)],
                    d_out.at[c, pl.ds(s * STRIPE, STRIPE)])
    pltpu.sync_copy(c_sh.at[pl.ds(s * STRIPE, STRIPE)],
                    c_out.at[c, pl.ds(s * STRIPE, STRIPE)])


@functools.partial(
    pl.kernel,
    mesh=plsc.VectorSubcoreMesh(**_MESH),
    compiler_params=pltpu.CompilerParams(needs_layout_passes=False),
    out_type=(
        jax.ShapeDtypeStruct((EP,), jnp.float32),
        jax.ShapeDtypeStruct((N2,), jnp.float32),
    ),
    scratch_types=[
        pltpu.VMEM((N2,), jnp.float32),
        pltpu.VMEM((N2,), jnp.float32),
        pltpu.VMEM((N2,), jnp.float32),
        pltpu.VMEM((N2 // NW,), jnp.float32),
        [pltpu.VMEM((KS * 2,), jnp.int32) for _ in range(3)],
        pltpu.VMEM((KS,), jnp.float32),
        [pltpu.SemaphoreType.DMA for _ in range(3)],
    ],
)
def _coeff_pass(ed2_hbm, dpart, cpart, coeff_out, selfc_out,
                dloc, cloc, tmp, selfv, eb, outv, sem_i):
    c = lax.axis_index("c")
    s = lax.axis_index("s")
    wid = s * NC + c

    base0 = (c * (EP // NC) + s * ET) * 2

    def istart(i, b):
        pltpu.async_copy(ed2_hbm.at[pl.ds(base0 + i * KS * 2, KS * 2)],
                         eb[b], sem_i[b])

    def iwait(i, b):
        pltpu.make_async_copy(ed2_hbm.at[pl.ds(base0 + i * KS * 2, KS * 2)],
                              eb[b], sem_i[b]).wait()

    istart(0, 0)
    istart(1, 1)

    pltpu.sync_copy(dpart.at[0], dloc)
    pltpu.sync_copy(dpart.at[1], tmp)

    def comb_d(j, _):
        sl = pl.ds(j * 16, 16)
        dloc[sl] = dloc[sl] + tmp[sl] + jnp.float32(EXP1)
        return 0
    lax.fori_loop(0, N2 // 16, comb_d, 0, unroll=4)

    pltpu.sync_copy(cpart.at[0], cloc)
    pltpu.sync_copy(cpart.at[1], tmp)

    def comb_c(j, _):
        sl = pl.ds(j * 16, 16)
        cloc[sl] = cloc[sl] + tmp[sl] + jnp.float32(1.0)
        return 0
    lax.fori_loop(0, N2 // 16, comb_c, 0, unroll=4)

    nper = N2 // NW
    nbase = wid * nper

    def selfc_body(j, _):
        sl = pl.ds(j * 16, 16)
        gl = pl.ds(nbase + j * 16, 16)
        selfv[sl] = jnp.float32(EXP1) / (dloc[gl] * cloc[gl])
        return 0
    lax.fori_loop(0, nper // 16, selfc_body, 0)
    pltpu.sync_copy(selfv, selfc_out.at[pl.ds(nbase, nper)])

    cbase0 = c * (EP // NC) + s * ET
    iot2 = _iota16() * 2
    for i in range(NCHS):
        b = i % 3
        iwait(i, b)

        def compute(g, _):
            sl = pl.ds(g * 16, 16)
            idx = iot2 + g * 32
            dst16 = plsc.load_gather(eb[b], [idx])
            w16 = plsc.bitcast(plsc.load_gather(eb[b], [idx + 1]),
                               jnp.float32)
            dd = plsc.load_gather(dloc, [dst16])
            cc = plsc.load_gather(cloc, [dst16])
            outv[sl] = jnp.exp(w16) / (dd * cc)
            return 0
        lax.fori_loop(0, KS // 16, compute, 0, unroll=2)
        pltpu.sync_copy(outv, coeff_out.at[pl.ds(cbase0 + i * KS, KS)])
        if i + 2 < NCHS:
            istart(i + 2, (i + 2) % 3)


@functools.partial(
    pl.kernel,
    mesh=plsc.VectorSubcoreMesh(**_MESH),
    compiler_params=pltpu.CompilerParams(needs_layout_passes=False),
    out_type=jax.ShapeDtypeStruct((NC, N2, D), jnp.float32),
    scratch_types=[
        [pltpu.VMEM((K * 3,), jnp.int32) for _ in range(NBUF)],
        [[pltpu.VMEM((16,), jnp.int32) for _ in range(K // 16)]
         for _ in range(NBUF)],
        [pltpu.VMEM((K,), jnp.int32) for _ in range(NBUF)],
        [pltpu.VMEM((K, D), jnp.float32) for _ in range(NBUF)],
        pltpu.VMEM_SHARED((N2, D), jnp.float32),
        [pltpu.SemaphoreType.DMA for _ in range(NBUF)],
        [pltpu.SemaphoreType.DMA for _ in range(NBUF)],
        [pltpu.SemaphoreType.DMA for _ in range(NBUF)],
    ],
)
def _row_pass(ed3_hbm, xl_hbm, acc_out,
              ebuf, srcv, dstv, rows, acc_sh, sem_i, sem_g, sem_s):
    c = lax.axis_index("c")
    s = lax.axis_index("s")

    def zrow(j, _):
        for q in range(D // 16):
            rows[0][j, pl.ds(q * 16, 16)] = jnp.zeros((16,), jnp.float32)
        return 0
    lax.fori_loop(0, K, zrow, 0)
    for z in range(STRIPE // K):
        pltpu.sync_copy(rows[0], acc_sh.at[pl.ds(s * STRIPE + z * K, K)])
    plsc.subcore_barrier()

    base0 = (c * (EP // NC) + s * ET) * 3
    iot3 = _iota16() * 3

    def istart(i, b):
        pltpu.async_copy(ed3_hbm.at[pl.ds(base0 + i * K * 3, K * 3)],
                         ebuf[b], sem_i[b])

    def iwait(i, b):
        pltpu.make_async_copy(ed3_hbm.at[pl.ds(base0 + i * K * 3, K * 3)],
                              ebuf[b], sem_i[b]).wait()

    def deint(b):
        for g in range(K // 16):
            idx = iot3 + g * 48
            sl = pl.ds(g * 16, 16)
            srcv[b][g][...] = plsc.load_gather(ebuf[b], [idx])
            dstv[b][sl] = plsc.load_gather(ebuf[b], [idx + 1])

    def gstart(b):
        for g in range(K // 16):
            pltpu.async_copy(xl_hbm.at[srcv[b][g]],
                             rows[b].at[pl.ds(g * 16, 16)], sem_g[b])

    def gwait(b):
        for g in range(K // 16):
            pltpu.make_async_copy(xl_hbm.at[srcv[b][g]],
                                  rows[b].at[pl.ds(g * 16, 16)],
                                  sem_g[b]).wait()

    istart(0, 0)
    istart(1, 1)
    istart(2, 2)
    iwait(0, 0)
    deint(0)
    gstart(0)
    iwait(1, 1)
    deint(1)
    gstart(1)

    def outer(i0, _):
        for bb in range(NBUF):
            b = bb
            b2 = (bb + 2) % NBUF
            b3 = (bb + 3) % NBUF
            bm1 = (bb - 1) % NBUF
            i = i0 * NBUF + bb
            gwait(b)

            def scale(j, _):
                cb = plsc.bitcast(plsc.load_gather(
                    ebuf[b], [jnp.full((16,), j * 3 + 2, jnp.int32)]),
                    jnp.float32)
                for q in range(D // 16):
                    sl = pl.ds(q * 16, 16)
                    rows[b][j, sl] = rows[b][j, sl] * cb
                return 0
            lax.fori_loop(0, K, scale, 0, unroll=4)
            pltpu.async_copy(rows[b], acc_sh.at[dstv[b]], sem_s[b], add=True)

            def wait_prev():
                pltpu.make_async_copy(rows[bm1], acc_sh.at[dstv[bm1]],
                                      sem_s[bm1]).wait()
            if bb == 0:
                @pl.when(i0 > 0)
                def _():
                    wait_prev()
            else:
                wait_prev()

            def issue_gather():
                iwait(i + 2, b2)
                deint(b2)
                gstart(b2)
            if bb < 2:
                issue_gather()
            else:
                @pl.when(i0 < NCHUNK // NBUF - 1)
                def _():
                    issue_gather()

            def issue_rec():
                istart(i + 3, b3)
            if bb == 0:
                issue_rec()
            else:
                @pl.when(i0 < NCHUNK // NBUF - 1)
                def _():
                    issue_rec()
        return 0
    lax.fori_loop(0, NCHUNK // NBUF, outer, 0)
    pltpu.make_async_copy(rows[(NCHUNK - 1) % NBUF],
                          acc_sh.at[dstv[(NCHUNK - 1) % NBUF]],
                          sem_s[(NCHUNK - 1) % NBUF]).wait()
    plsc.subcore_barrier()
    pltpu.sync_copy(acc_sh.at[pl.ds(s * STRIPE, STRIPE)],
                    acc_out.at[c, pl.ds(s * STRIPE, STRIPE)])


def _mm_bias(xin, w, b):
    m = xin.shape[0]
    bm = 1000

    def body(x_ref, w_ref, b_ref, o_ref):
        o_ref[...] = jnp.dot(x_ref[...], w_ref[...],
                             preferred_element_type=jnp.float32) + b_ref[...]

    return pl.pallas_call(
        body,
        grid=(m // bm,),
        in_specs=[
            pl.BlockSpec((bm, D), lambda i: (i, 0)),
            pl.BlockSpec((D, D), lambda i: (0, 0)),
            pl.BlockSpec((1, D), lambda i: (0, 0)),
        ],
        out_specs=pl.BlockSpec((bm, D), lambda i: (i, 0)),
        out_shape=jax.ShapeDtypeStruct((m, D), jnp.float32),
    )(xin, w, b.reshape(1, D))


def _post(acc0, acc1, selfc, xl, w, relu):
    m = xl.shape[0]
    bm = 1000

    def body(a0_ref, a1_ref, sc_ref, x_ref, w_ref, o_ref):
        aggr = a0_ref[...] + a1_ref[...] + sc_ref[...] * x_ref[...]
        o = jnp.dot(aggr, w_ref[...], preferred_element_type=jnp.float32)
        if relu:
            o = jnp.maximum(o, jnp.float32(0.0))
        o_ref[...] = o

    return pl.pallas_call(
        body,
        grid=(m // bm,),
        in_specs=[
            pl.BlockSpec((bm, D), lambda i: (i, 0)),
            pl.BlockSpec((bm, D), lambda i: (i, 0)),
            pl.BlockSpec((bm, 1), lambda i: (i, 0)),
            pl.BlockSpec((bm, D), lambda i: (i, 0)),
            pl.BlockSpec((D, D), lambda i: (0, 0)),
        ],
        out_specs=pl.BlockSpec((bm, D), lambda i: (i, 0)),
        out_shape=jax.ShapeDtypeStruct((m, D), jnp.float32),
    )(acc0, acc1, selfc, xl, w)


def kernel(x, edge_index, edge_weight, lin1_w, lin1_b, w1, lin2_w, lin2_b, w2):
    src = edge_index[0]
    dst = edge_index[1]
    pad = EP - E
    zpad = jnp.zeros((pad,), jnp.int32)
    src_p = jnp.concatenate([src, zpad])
    dst_p = jnp.concatenate([dst, zpad])
    ew_p = jnp.concatenate([edge_weight,
                            jnp.full((pad,), -100.0, jnp.float32)])
    ewb = lax.bitcast_convert_type(ew_p, jnp.int32)
    ed2 = jnp.stack([dst_p, ewb], axis=1).reshape(-1)

    dpart, cpart = _scalar_pass(ed2)
    coeff, selfc = _coeff_pass(ed2, dpart, cpart)
    selfc = selfc[:N].reshape(N, 1)
    ed3 = jnp.stack(
        [src_p, dst_p, lax.bitcast_convert_type(coeff, jnp.int32)],
        axis=1).reshape(-1)

    xl1 = _mm_bias(x, lin1_w, lin1_b)
    acc1 = _row_pass(ed3, xl1)
    h = _post(acc1[0, :N], acc1[1, :N], selfc, xl1, w1, relu=True)

    xl2 = _mm_bias(h, lin2_w, lin2_b)
    acc2 = _row_pass(ed3, xl2)
    out = _post(acc2[0, :N], acc2[1, :N], selfc, xl2, w2, relu=False)
    return out


# speedup vs baseline: 1.5601x; 1.5601x over previous
import functools

import numpy as np
import jax
import jax.numpy as jnp
from jax import lax
from jax.experimental import pallas as pl
from jax.experimental.pallas import tpu as pltpu
from jax.experimental.pallas import tpu_sc as plsc

N = 10000
D = 128
E = 320000
NC = 2
NS = 16
NW = NC * NS
EP = 327680
ET = EP // NW
K = 64
NCHUNK = ET // K
NBUF = 4
KS = 512
NCHS = ET // KS
N2 = 10240
STRIPE = N2 // NS
EXP1 = float(np.exp(np.float32(1.0)))

_MESH = dict(core_axis_name="c", subcore_axis_name="s")


def _iota16():
    return lax.iota(jnp.int32, 16)


@functools.partial(
    pl.kernel,
    mesh=plsc.VectorSubcoreMesh(**_MESH),
    compiler_params=pltpu.CompilerParams(needs_layout_passes=False),
    out_type=(
        jax.ShapeDtypeStruct((EP,), jnp.float32),
        jax.ShapeDtypeStruct((N2,), jnp.float32),
    ),
    scratch_types=[
        pltpu.VMEM((N2,), jnp.float32),
        pltpu.VMEM((N2,), jnp.float32),
        [pltpu.VMEM((KS * 2,), jnp.int32) for _ in range(3)],
        pltpu.VMEM((KS,), jnp.int32),
        pltpu.VMEM((KS,), jnp.float32),
        pltpu.VMEM((KS,), jnp.float32),
        pltpu.VMEM((KS,), jnp.float32),
        pltpu.VMEM((STRIPE,), jnp.float32),
        pltpu.VMEM_SHARED((N2,), jnp.float32),
        pltpu.VMEM_SHARED((N2,), jnp.float32),
        [pltpu.SemaphoreType.DMA for _ in range(3)],
    ],
)
def _coeff_pass(ed2_hbm, coeff_out, selfc_out,
                dloc, cloc, eb, dstb, valsb, onesb, outv, zv,
                d_sh, c_sh, sem_i):
    c = lax.axis_index("c")
    s = lax.axis_index("s")
    wid = s * NC + c

    def zbody(j, _):
        zv[pl.ds(j * 16, 16)] = jnp.zeros((16,), jnp.float32)
        return 0
    lax.fori_loop(0, STRIPE // 16, zbody, 0)
    pltpu.sync_copy(zv, d_sh.at[pl.ds(s * STRIPE, STRIPE)])
    pltpu.sync_copy(zv, c_sh.at[pl.ds(s * STRIPE, STRIPE)])
    plsc.subcore_barrier()

    ET2 = EP // NS
    NCH2 = ET2 // KS
    p1base = s * ET2 * 2

    def istart1(i, b):
        pltpu.async_copy(ed2_hbm.at[pl.ds(p1base + i * KS * 2, KS * 2)],
                         eb[b], sem_i[b])

    def iwait1(i, b):
        pltpu.make_async_copy(ed2_hbm.at[pl.ds(p1base + i * KS * 2, KS * 2)],
                              eb[b], sem_i[b]).wait()

    istart1(0, 0)
    istart1(1, 1)
    iot2 = _iota16() * 2
    for i in range(NCH2):
        b = i % 3
        iwait1(i, b)

        def compute(g, _):
            sl = pl.ds(g * 16, 16)
            idx = iot2 + g * 32
            dstb[sl] = plsc.load_gather(eb[b], [idx])
            w16 = plsc.bitcast(plsc.load_gather(eb[b], [idx + 1]),
                               jnp.float32)
            valsb[sl] = jnp.exp(w16)
            onesb[sl] = jnp.where(w16 > jnp.float32(-50.0),
                                  jnp.float32(1.0), jnp.float32(0.0))
            return 0
        lax.fori_loop(0, KS // 16, compute, 0, unroll=2)
        pltpu.sync_copy(valsb, d_sh.at[dstb], add=True)
        pltpu.sync_copy(onesb, c_sh.at[dstb], add=True)
        if i + 2 < NCH2:
            istart1(i + 2, (i + 2) % 3)
    plsc.subcore_barrier()

    pltpu.sync_copy(d_sh, dloc)
    pltpu.sync_copy(c_sh, cloc)

    def comb(j, _):
        sl = pl.ds(j * 16, 16)
        dloc[sl] = dloc[sl] + jnp.float32(EXP1)
        cloc[sl] = cloc[sl] + jnp.float32(1.0)
        return 0
    lax.fori_loop(0, N2 // 16, comb, 0, unroll=4)

    nper = N2 // NW
    nbase = wid * nper

    def selfc_body(j, _):
        sl = pl.ds(nbase + j * 16, 16)
        outv[pl.ds(j * 16, 16)] = (jnp.float32(EXP1)
                                   / (dloc[sl] * cloc[sl]))
        return 0
    lax.fori_loop(0, nper // 16, selfc_body, 0)
    pltpu.sync_copy(outv.at[pl.ds(0, nper)], selfc_out.at[pl.ds(nbase, nper)])

    cbase0 = c * (EP // NC) + s * ET
    p2base = cbase0 * 2

    def istart2(i, b):
        pltpu.async_copy(ed2_hbm.at[pl.ds(p2base + i * KS * 2, KS * 2)],
                         eb[b], sem_i[b])

    def iwait2(i, b):
        pltpu.make_async_copy(ed2_hbm.at[pl.ds(p2base + i * KS * 2, KS * 2)],
                              eb[b], sem_i[b]).wait()

    istart2(0, 0)
    istart2(1, 1)
    for i in range(NCHS):
        b = i % 3
        iwait2(i, b)

        def compute2(g, _):
            sl = pl.ds(g * 16, 16)
            idx = iot2 + g * 32
            dst16 = plsc.load_gather(eb[b], [idx])
            w16 = plsc.bitcast(plsc.load_gather(eb[b], [idx + 1]),
                               jnp.float32)
            dd = plsc.load_gather(dloc, [dst16])
            cc = plsc.load_gather(cloc, [dst16])
            outv[sl] = jnp.exp(w16) / (dd * cc)
            return 0
        lax.fori_loop(0, KS // 16, compute2, 0, unroll=2)
        pltpu.sync_copy(outv, coeff_out.at[pl.ds(cbase0 + i * KS, KS)])
        if i + 2 < NCHS:
            istart2(i + 2, (i + 2) % 3)


NBUF = 4

@functools.partial(
    pl.kernel,
    mesh=plsc.VectorSubcoreMesh(**_MESH),
    compiler_params=pltpu.CompilerParams(needs_layout_passes=False),
    out_type=jax.ShapeDtypeStruct((NC, N2, D), jnp.float32),
    scratch_types=[
        [pltpu.VMEM((K * 3,), jnp.int32) for _ in range(NBUF)],
        [pltpu.VMEM((K,), jnp.int32) for _ in range(NBUF)],
        [pltpu.VMEM((K,), jnp.int32) for _ in range(NBUF)],
        [pltpu.VMEM((K, D), jnp.float32) for _ in range(NBUF)],
        pltpu.VMEM_SHARED((N2, D), jnp.float32),
        [pltpu.SemaphoreType.DMA for _ in range(NBUF)],
        [pltpu.SemaphoreType.DMA for _ in range(NBUF)],
    ],
)
def _row_pass(ed3_hbm, xl_hbm, acc_out,
              ebuf, srcv, dstv, rows, acc_sh, sem_i, sem_g):
    c = lax.axis_index("c")
    s = lax.axis_index("s")

    def zrow(j, _):
        for q in range(D // 16):
            rows[0][j, pl.ds(q * 16, 16)] = jnp.zeros((16,), jnp.float32)
        return 0
    lax.fori_loop(0, K, zrow, 0)
    for z in range(STRIPE // K):
        pltpu.sync_copy(rows[0], acc_sh.at[pl.ds(s * STRIPE + z * K, K)])
    plsc.subcore_barrier()

    base0 = (c * (EP // NC) + s * ET) * 3
    iot3 = _iota16() * 3

    def istart(i, b):
        pltpu.async_copy(ed3_hbm.at[pl.ds(base0 + i * K * 3, K * 3)],
                         ebuf[b], sem_i[b])

    def iwait(i, b):
        pltpu.make_async_copy(ed3_hbm.at[pl.ds(base0 + i * K * 3, K * 3)],
                              ebuf[b], sem_i[b]).wait()

    def deint(b):
        for g in range(K // 16):
            idx = iot3 + g * 48
            sl = pl.ds(g * 16, 16)
            srcv[b][sl] = plsc.load_gather(ebuf[b], [idx])
            dstv[b][sl] = plsc.load_gather(ebuf[b], [idx + 1])

    def gstart(b):
        pltpu.async_copy(xl_hbm.at[srcv[b]], rows[b], sem_g[b])

    def gwait(b):
        pltpu.make_async_copy(xl_hbm.at[srcv[b]], rows[b], sem_g[b]).wait()

    istart(0, 0)
    istart(1, 1)
    istart(2, 2)
    iwait(0, 0)
    deint(0)
    gstart(0)
    iwait(1, 1)
    deint(1)
    gstart(1)

    def outer(i0, _):
        for bb in range(NBUF):
            b = bb
            b2 = (bb + 2) % NBUF
            b3 = (bb + 3) % NBUF
            i = i0 * NBUF + bb
            gwait(b)

            def scale(j, _):
                cb = plsc.bitcast(plsc.load_gather(
                    ebuf[b], [jnp.full((16,), j * 3 + 2, jnp.int32)]),
                    jnp.float32)
                for q in range(D // 16):
                    sl = pl.ds(q * 16, 16)
                    rows[b][j, sl] = rows[b][j, sl] * cb
                return 0
            lax.fori_loop(0, K, scale, 0, unroll=4)
            pltpu.sync_copy(rows[b], acc_sh.at[dstv[b]], add=True)

            def issue_gather():
                iwait(i + 2, b2)
                deint(b2)
                gstart(b2)
            if bb < 2:
                issue_gather()
            else:
                @pl.when(i0 < NCHUNK // NBUF - 1)
                def _():
                    issue_gather()

            def issue_rec():
                istart(i + 3, b3)
            if bb == 0:
                issue_rec()
            else:
                @pl.when(i0 < NCHUNK // NBUF - 1)
                def _():
                    issue_rec()
        return 0
    lax.fori_loop(0, NCHUNK // NBUF, outer, 0)
    plsc.subcore_barrier()
    pltpu.sync_copy(acc_sh.at[pl.ds(s * STRIPE, STRIPE)],
                    acc_out.at[c, pl.ds(s * STRIPE, STRIPE)])


def _mm_bias(xin, w, b):
    m = xin.shape[0]
    bm = 1000

    def body(x_ref, w_ref, b_ref, o_ref):
        o_ref[...] = jnp.dot(x_ref[...], w_ref[...],
                             preferred_element_type=jnp.float32) + b_ref[...]

    return pl.pallas_call(
        body,
        grid=(m // bm,),
        in_specs=[
            pl.BlockSpec((bm, D), lambda i: (i, 0)),
            pl.BlockSpec((D, D), lambda i: (0, 0)),
            pl.BlockSpec((1, D), lambda i: (0, 0)),
        ],
        out_specs=pl.BlockSpec((bm, D), lambda i: (i, 0)),
        out_shape=jax.ShapeDtypeStruct((m, D), jnp.float32),
    )(xin, w, b.reshape(1, D))


def _post(acc0, acc1, selfc, xl, w, relu):
    m = xl.shape[0]
    bm = 1000

    def body(a0_ref, a1_ref, sc_ref, x_ref, w_ref, o_ref):
        aggr = a0_ref[...] + a1_ref[...] + sc_ref[...] * x_ref[...]
        o = jnp.dot(aggr, w_ref[...], preferred_element_type=jnp.float32)
        if relu:
            o = jnp.maximum(o, jnp.float32(0.0))
        o_ref[...] = o

    return pl.pallas_call(
        body,
        grid=(m // bm,),
        in_specs=[
            pl.BlockSpec((bm, D), lambda i: (i, 0)),
            pl.BlockSpec((bm, D), lambda i: (i, 0)),
            pl.BlockSpec((bm, 1), lambda i: (i, 0)),
            pl.BlockSpec((bm, D), lambda i: (i, 0)),
            pl.BlockSpec((D, D), lambda i: (0, 0)),
        ],
        out_specs=pl.BlockSpec((bm, D), lambda i: (i, 0)),
        out_shape=jax.ShapeDtypeStruct((m, D), jnp.float32),
    )(acc0, acc1, selfc, xl, w)


def kernel(x, edge_index, edge_weight, lin1_w, lin1_b, w1, lin2_w, lin2_b, w2):
    src = edge_index[0]
    dst = edge_index[1]
    pad = EP - E
    spread = jnp.arange(pad, dtype=jnp.int32) % N
    src_p = jnp.concatenate([src, spread])
    dst_p = jnp.concatenate([dst, spread])
    ew_p = jnp.concatenate([edge_weight,
                            jnp.full((pad,), -100.0, jnp.float32)])
    ewb = lax.bitcast_convert_type(ew_p, jnp.int32)
    ed2 = jnp.stack([dst_p, ewb], axis=1).reshape(-1)

    coeff, selfc = _coeff_pass(ed2)
    selfc = selfc[:N].reshape(N, 1)
    ed3 = jnp.stack(
        [src_p, dst_p, lax.bitcast_convert_type(coeff, jnp.int32)],
        axis=1).reshape(-1)

    xl1 = _mm_bias(x, lin1_w, lin1_b)
    acc1 = _row_pass(ed3, xl1)
    h = _post(acc1[0, :N], acc1[1, :N], selfc, xl1, w1, relu=True)

    xl2 = _mm_bias(h, lin2_w, lin2_b)
    acc2 = _row_pass(ed3, xl2)
    out = _post(acc2[0, :N], acc2[1, :N], selfc, xl2, w2, relu=False)
    return out


# speedup vs baseline: 1.5680x; 1.0051x over previous
import functools

import numpy as np
import jax
import jax.numpy as jnp
from jax import lax
from jax.experimental import pallas as pl
from jax.experimental.pallas import tpu as pltpu
from jax.experimental.pallas import tpu_sc as plsc

N = 10000
D = 128
E = 320000
NC = 2
NS = 16
NW = NC * NS
EP = 327680
ET = EP // NW
K = 64
NCHUNK = ET // K
KS = 512
NCHS = ET // KS
N2 = 10240
STRIPE = N2 // NS
EXP1 = float(np.exp(np.float32(1.0)))

_MESH = dict(core_axis_name="c", subcore_axis_name="s")


def _iota16():
    return lax.iota(jnp.int32, 16)


def _row_phase(c, s, edsd_hbm, xl_hbm, acc_out,
               ebuf, srcv, dstv, rows, coeffloc, acc_sh, sem_i, sem_g):
    for r in range(2):
        def zrow(j, _, r=r):
            for q in range(D // 16):
                rows[r][j, pl.ds(q * 16, 16)] = jnp.zeros((16,), jnp.float32)
            return 0
        lax.fori_loop(0, K, zrow, 0)
    for z in range(STRIPE // K):
        pltpu.sync_copy(rows[z % 2], acc_sh.at[pl.ds(s * STRIPE + z * K, K)])
    plsc.subcore_barrier()

    base0 = (c * (EP // NC) + s * ET) * 2
    iot2 = _iota16() * 2

    def istart(i, b):
        pltpu.async_copy(edsd_hbm.at[pl.ds(base0 + i * K * 2, K * 2)],
                         ebuf[b], sem_i[b])

    def iwait(i, b):
        pltpu.make_async_copy(edsd_hbm.at[pl.ds(base0 + i * K * 2, K * 2)],
                              ebuf[b], sem_i[b]).wait()

    def deint(b):
        for g in range(K // 16):
            idx = iot2 + g * 32
            sl = pl.ds(g * 16, 16)
            srcv[b][sl] = plsc.load_gather(ebuf[b], [idx])
            dstv[b][sl] = plsc.load_gather(ebuf[b], [idx + 1])

    istart(0, 0)
    istart(1, 1)
    iwait(0, 0)
    deint(0)
    pltpu.async_copy(xl_hbm.at[srcv[0]], rows[0], sem_g[0])
    istart(2, 2)

    def outer(i0, _):
        for bb in range(4):
            i = i0 * 4 + bb
            b = bb
            rb = bb % 2
            b1 = (bb + 1) % 4
            b3 = (bb + 3) % 4

            def issue_next():
                iwait(i + 1, b1)
                deint(b1)
                pltpu.async_copy(xl_hbm.at[srcv[b1]], rows[1 - rb],
                                 sem_g[1 - rb])
            if bb < 3:
                issue_next()
            else:
                @pl.when(i0 < NCHUNK // 4 - 1)
                def _():
                    issue_next()

            def issue_rec():
                istart(i + 3, b3)
            if bb == 0:
                issue_rec()
            else:
                @pl.when(i0 < NCHUNK // 4 - 1)
                def _():
                    issue_rec()
            pltpu.make_async_copy(xl_hbm.at[srcv[b]], rows[rb],
                                  sem_g[rb]).wait()

            def scale(j, _):
                cb = plsc.load_gather(
                    coeffloc, [jnp.full((16,), i * K + j, jnp.int32)])
                for q in range(D // 16):
                    sl = pl.ds(q * 16, 16)
                    rows[rb][j, sl] = rows[rb][j, sl] * cb
                return 0
            lax.fori_loop(0, K, scale, 0, unroll=4)
            pltpu.sync_copy(rows[rb], acc_sh.at[dstv[b]], add=True)
        return 0
    lax.fori_loop(0, NCHUNK // 4, outer, 0)
    plsc.subcore_barrier()
    pltpu.sync_copy(acc_sh.at[pl.ds(s * STRIPE, STRIPE)],
                    acc_out.at[c, pl.ds(s * STRIPE, STRIPE)])


def _row_scratch():
    return [
        [pltpu.VMEM((K * 2,), jnp.int32) for _ in range(4)],
        [pltpu.VMEM((K,), jnp.int32) for _ in range(4)],
        [pltpu.VMEM((K,), jnp.int32) for _ in range(4)],
        [pltpu.VMEM((K, D), jnp.float32) for _ in range(2)],
        pltpu.VMEM((ET,), jnp.float32),
        pltpu.VMEM_SHARED((N2, D), jnp.float32),
        [pltpu.SemaphoreType.DMA for _ in range(4)],
        [pltpu.SemaphoreType.DMA for _ in range(2)],
    ]


@functools.partial(
    pl.kernel,
    mesh=plsc.VectorSubcoreMesh(**_MESH),
    compiler_params=pltpu.CompilerParams(needs_layout_passes=False),
    out_type=(
        jax.ShapeDtypeStruct((NC, N2, D), jnp.float32),
        jax.ShapeDtypeStruct((EP,), jnp.float32),
        jax.ShapeDtypeStruct((N2,), jnp.float32),
    ),
    scratch_types=[
        pltpu.VMEM((N2,), jnp.float32),
        [pltpu.VMEM((KS * 2,), jnp.int32) for _ in range(3)],
        pltpu.VMEM((KS,), jnp.int32),
        pltpu.VMEM((KS,), jnp.float32),
        pltpu.VMEM((KS,), jnp.float32),
        pltpu.VMEM((KS,), jnp.float32),
        pltpu.VMEM((STRIPE,), jnp.float32),
        pltpu.VMEM_SHARED((N2,), jnp.float32),
        pltpu.VMEM_SHARED((N2,), jnp.float32),
        [pltpu.SemaphoreType.DMA for _ in range(3)],
    ] + _row_scratch(),
)
def _fused_pass(ed2_hbm, edsd_hbm, xl_hbm, acc_out, coeff_out, selfc_out,
                prodloc, eb, dstb, valsb, onesb, outv, zv, d_sh, c_sh, sem_s,
                ebuf, srcv, dstv, rows, coeffloc, acc_sh, sem_i, sem_g):
    c = lax.axis_index("c")
    s = lax.axis_index("s")
    wid = s * NC + c

    def zbody(j, _):
        zv[pl.ds(j * 16, 16)] = jnp.zeros((16,), jnp.float32)
        return 0
    lax.fori_loop(0, STRIPE // 16, zbody, 0)
    pltpu.sync_copy(zv, d_sh.at[pl.ds(s * STRIPE, STRIPE)])
    pltpu.sync_copy(zv, c_sh.at[pl.ds(s * STRIPE, STRIPE)])
    plsc.subcore_barrier()

    ET2 = EP // NS
    NCH2 = ET2 // KS
    p1base = s * ET2 * 2

    def istart1(i, b):
        pltpu.async_copy(ed2_hbm.at[pl.ds(p1base + i * KS * 2, KS * 2)],
                         eb[b], sem_s[b])

    def iwait1(i, b):
        pltpu.make_async_copy(ed2_hbm.at[pl.ds(p1base + i * KS * 2, KS * 2)],
                              eb[b], sem_s[b]).wait()

    istart1(0, 0)
    istart1(1, 1)
    iot2 = _iota16() * 2
    for i in range(NCH2):
        b = i % 3
        iwait1(i, b)

        def compute(g, _):
            sl = pl.ds(g * 16, 16)
            idx = iot2 + g * 32
            dstb[sl] = plsc.load_gather(eb[b], [idx])
            w16 = plsc.bitcast(plsc.load_gather(eb[b], [idx + 1]),
                               jnp.float32)
            valsb[sl] = jnp.exp(w16)
            onesb[sl] = jnp.where(w16 > jnp.float32(-50.0),
                                  jnp.float32(1.0), jnp.float32(0.0))
            return 0
        lax.fori_loop(0, KS // 16, compute, 0, unroll=2)
        pltpu.sync_copy(valsb, d_sh.at[dstb], add=True)
        pltpu.sync_copy(onesb, c_sh.at[dstb], add=True)
        if i + 2 < NCH2:
            istart1(i + 2, (i + 2) % 3)
    plsc.subcore_barrier()

    pltpu.sync_copy(d_sh, prodloc)
    pltpu.sync_copy(c_sh, coeffloc)

    def comb(j, _):
        sl = pl.ds(j * 16, 16)
        prodloc[sl] = ((prodloc[sl] + jnp.float32(EXP1))
                       * (coeffloc[sl] + jnp.float32(1.0)))
        return 0
    lax.fori_loop(0, N2 // 16, comb, 0, unroll=4)

    nper = N2 // NW
    nbase = wid * nper

    def selfc_body(j, _):
        outv[pl.ds(j * 16, 16)] = (jnp.float32(EXP1)
                                   / prodloc[pl.ds(nbase + j * 16, 16)])
        return 0
    lax.fori_loop(0, nper // 16, selfc_body, 0)
    pltpu.sync_copy(outv.at[pl.ds(0, nper)], selfc_out.at[pl.ds(nbase, nper)])

    cbase0 = c * (EP // NC) + s * ET
    p2base = cbase0 * 2

    def istart2(i, b):
        pltpu.async_copy(ed2_hbm.at[pl.ds(p2base + i * KS * 2, KS * 2)],
                         eb[b], sem_s[b])

    def iwait2(i, b):
        pltpu.make_async_copy(ed2_hbm.at[pl.ds(p2base + i * KS * 2, KS * 2)],
                              eb[b], sem_s[b]).wait()

    istart2(0, 0)
    istart2(1, 1)
    for i in range(NCHS):
        b = i % 3
        iwait2(i, b)

        def compute2(g, _):
            sl = pl.ds(g * 16, 16)
            idx = iot2 + g * 32
            dst16 = plsc.load_gather(eb[b], [idx])
            w16 = plsc.bitcast(plsc.load_gather(eb[b], [idx + 1]),
                               jnp.float32)
            outv[sl] = jnp.exp(w16) / plsc.load_gather(prodloc, [dst16])
            return 0
        lax.fori_loop(0, KS // 16, compute2, 0, unroll=2)

        def copy_local(g, _):
            coeffloc[pl.ds(i * KS + g * 16, 16)] = outv[pl.ds(g * 16, 16)]
            return 0
        lax.fori_loop(0, KS // 16, copy_local, 0, unroll=4)
        pltpu.sync_copy(outv, coeff_out.at[pl.ds(cbase0 + i * KS, KS)])
        if i + 2 < NCHS:
            istart2(i + 2, (i + 2) % 3)

    _row_phase(c, s, edsd_hbm, xl_hbm, acc_out,
               ebuf, srcv, dstv, rows, coeffloc, acc_sh, sem_i, sem_g)


@functools.partial(
    pl.kernel,
    mesh=plsc.VectorSubcoreMesh(**_MESH),
    compiler_params=pltpu.CompilerParams(needs_layout_passes=False),
    out_type=jax.ShapeDtypeStruct((NC, N2, D), jnp.float32),
    scratch_types=_row_scratch(),
)
def _row_pass(edsd_hbm, coeff_hbm, xl_hbm, acc_out,
              ebuf, srcv, dstv, rows, coeffloc, acc_sh, sem_i, sem_g):
    c = lax.axis_index("c")
    s = lax.axis_index("s")
    cbase0 = c * (EP // NC) + s * ET
    pltpu.sync_copy(coeff_hbm.at[pl.ds(cbase0, ET)], coeffloc)
    _row_phase(c, s, edsd_hbm, xl_hbm, acc_out,
               ebuf, srcv, dstv, rows, coeffloc, acc_sh, sem_i, sem_g)


def _mm_bias(xin, w, b):
    m = xin.shape[0]
    bm = 1000

    def body(x_ref, w_ref, b_ref, o_ref):
        o_ref[...] = jnp.dot(x_ref[...], w_ref[...],
                             preferred_element_type=jnp.float32) + b_ref[...]

    return pl.pallas_call(
        body,
        grid=(m // bm,),
        in_specs=[
            pl.BlockSpec((bm, D), lambda i: (i, 0)),
            pl.BlockSpec((D, D), lambda i: (0, 0)),
            pl.BlockSpec((1, D), lambda i: (0, 0)),
        ],
        out_specs=pl.BlockSpec((bm, D), lambda i: (i, 0)),
        out_shape=jax.ShapeDtypeStruct((m, D), jnp.float32),
    )(xin, w, b.reshape(1, D))


def _post(acc0, acc1, selfc, xl, w, relu):
    m = xl.shape[0]
    bm = 1000

    def body(a0_ref, a1_ref, sc_ref, x_ref, w_ref, o_ref):
        aggr = a0_ref[...] + a1_ref[...] + sc_ref[...] * x_ref[...]
        o = jnp.dot(aggr, w_ref[...], preferred_element_type=jnp.float32)
        if relu:
            o = jnp.maximum(o, jnp.float32(0.0))
        o_ref[...] = o

    return pl.pallas_call(
        body,
        grid=(m // bm,),
        in_specs=[
            pl.BlockSpec((bm, D), lambda i: (i, 0)),
            pl.BlockSpec((bm, D), lambda i: (i, 0)),
            pl.BlockSpec((bm, 1), lambda i: (i, 0)),
            pl.BlockSpec((bm, D), lambda i: (i, 0)),
            pl.BlockSpec((D, D), lambda i: (0, 0)),
        ],
        out_specs=pl.BlockSpec((bm, D), lambda i: (i, 0)),
        out_shape=jax.ShapeDtypeStruct((m, D), jnp.float32),
    )(acc0, acc1, selfc, xl, w)


def kernel(x, edge_index, edge_weight, lin1_w, lin1_b, w1, lin2_w, lin2_b, w2):
    src = edge_index[0]
    dst = edge_index[1]
    pad = EP - E
    spread = jnp.arange(pad, dtype=jnp.int32) % N
    src_p = jnp.concatenate([src, spread])
    dst_p = jnp.concatenate([dst, spread])
    ew_p = jnp.concatenate([edge_weight,
                            jnp.full((pad,), -100.0, jnp.float32)])
    ewb = lax.bitcast_convert_type(ew_p, jnp.int32)
    ed2 = jnp.stack([dst_p, ewb], axis=1).reshape(-1)
    edsd = jnp.stack([src_p, dst_p], axis=1).reshape(-1)

    xl1 = _mm_bias(x, lin1_w, lin1_b)
    acc1, coeff, selfc = _fused_pass(ed2, edsd, xl1)
    selfc = selfc[:N].reshape(N, 1)
    h = _post(acc1[0, :N], acc1[1, :N], selfc, xl1, w1, relu=True)

    xl2 = _mm_bias(h, lin2_w, lin2_b)
    acc2 = _row_pass(edsd, coeff, xl2)
    out = _post(acc2[0, :N], acc2[1, :N], selfc, xl2, w2, relu=False)
    return out


# speedup vs baseline: 2.7829x; 1.7748x over previous
import functools

import numpy as np
import jax
import jax.numpy as jnp
from jax import lax
from jax.experimental import pallas as pl
from jax.experimental.pallas import tpu as pltpu
from jax.experimental.pallas import tpu_sc as plsc

N = 10000
D = 128
E = 320000
NC = 2
NS = 16
NW = NC * NS
N2 = 10240
STRIPE = N2 // NS
EP = 327680
EP2 = 344064
ETR = EP2 // NW
K = 64
NCHUNK = ETR // K
KS = 512
EXP1 = float(np.exp(np.float32(1.0)))

_MESH = dict(core_axis_name="c", subcore_axis_name="s")


def _iota16():
    return lax.iota(jnp.int32, 16)


def _row_phase(c, s, packed_hbm, xl_hbm, acc_out,
               ebuf, srcv, dstv, rows, coeffloc, acc_sh, sem_i, sem_g):
    for r in range(2):
        def zrow(j, _, r=r):
            for q in range(D // 16):
                rows[r][j, pl.ds(q * 16, 16)] = jnp.zeros((16,), jnp.float32)
            return 0
        lax.fori_loop(0, K, zrow, 0)
    for z in range(STRIPE // K):
        pltpu.sync_copy(rows[z % 2], acc_sh.at[pl.ds(s * STRIPE + z * K, K)])
    plsc.subcore_barrier()

    base0 = c * (EP2 // NC) + s * ETR

    def istart(i, b):
        pltpu.async_copy(packed_hbm.at[pl.ds(base0 + i * K, K)],
                         ebuf[b], sem_i[b])

    def iwait(i, b):
        pltpu.make_async_copy(packed_hbm.at[pl.ds(base0 + i * K, K)],
                              ebuf[b], sem_i[b]).wait()

    def deint(b):
        for g in range(K // 16):
            sl = pl.ds(g * 16, 16)
            v = ebuf[b][sl]
            srcv[b][sl] = lax.bitwise_and(v, jnp.int32(16383))
            dstv[b][sl] = lax.shift_right_logical(v, 14)

    istart(0, 0)
    istart(1, 1)
    iwait(0, 0)
    deint(0)
    pltpu.async_copy(xl_hbm.at[srcv[0]], rows[0], sem_g[0])
    istart(2, 2)

    def outer(i0, _):
        for bb in range(4):
            i = i0 * 4 + bb
            b = bb
            rb = bb % 2
            b1 = (bb + 1) % 4
            b3 = (bb + 3) % 4

            def issue_next():
                iwait(i + 1, b1)
                deint(b1)
                pltpu.async_copy(xl_hbm.at[srcv[b1]], rows[1 - rb],
                                 sem_g[1 - rb])
            if bb < 3:
                issue_next()
            else:
                @pl.when(i0 < NCHUNK // 4 - 1)
                def _():
                    issue_next()

            def issue_rec():
                istart(i + 3, b3)
            if bb == 0:
                issue_rec()
            else:
                @pl.when(i0 < NCHUNK // 4 - 1)
                def _():
                    issue_rec()
            pltpu.make_async_copy(xl_hbm.at[srcv[b]], rows[rb],
                                  sem_g[rb]).wait()

            def scale(j, _):
                cb = plsc.load_gather(
                    coeffloc, [jnp.full((16,), i * K + j, jnp.int32)])
                for q in range(D // 16):
                    sl = pl.ds(q * 16, 16)
                    rows[rb][j, sl] = rows[rb][j, sl] * cb
                return 0
            lax.fori_loop(0, K, scale, 0, unroll=4)
            pltpu.sync_copy(rows[rb], acc_sh.at[dstv[b]], add=True)
        return 0
    lax.fori_loop(0, NCHUNK // 4, outer, 0)
    plsc.subcore_barrier()
    pltpu.sync_copy(acc_sh.at[pl.ds(s * STRIPE, STRIPE)],
                    acc_out.at[c, pl.ds(s * STRIPE, STRIPE)])


def _row_scratch():
    return [
        [pltpu.VMEM((K,), jnp.int32) for _ in range(4)],
        [pltpu.VMEM((K,), jnp.int32) for _ in range(4)],
        [pltpu.VMEM((K,), jnp.int32) for _ in range(4)],
        [pltpu.VMEM((K, D), jnp.float32) for _ in range(2)],
        pltpu.VMEM((ETR,), jnp.float32),
        pltpu.VMEM_SHARED((N2, D), jnp.float32),
        [pltpu.SemaphoreType.DMA for _ in range(4)],
        [pltpu.SemaphoreType.DMA for _ in range(2)],
    ]


@functools.partial(
    pl.kernel,
    mesh=plsc.VectorSubcoreMesh(**_MESH),
    compiler_params=pltpu.CompilerParams(needs_layout_passes=False),
    out_type=(
        jax.ShapeDtypeStruct((NC, N2, D), jnp.float32),
        jax.ShapeDtypeStruct((EP2,), jnp.float32),
    ),
    scratch_types=[
        pltpu.VMEM((N2,), jnp.float32),
        [pltpu.VMEM((KS,), jnp.int32) for _ in range(3)],
        [pltpu.VMEM((KS,), jnp.float32) for _ in range(3)],
        pltpu.VMEM((KS,), jnp.float32),
        pltpu.VMEM((KS,), jnp.float32),
        pltpu.VMEM((KS,), jnp.float32),
        pltpu.VMEM((STRIPE,), jnp.float32),
        pltpu.VMEM_SHARED((N2,), jnp.float32),
        pltpu.VMEM_SHARED((N2,), jnp.float32),
        [pltpu.SemaphoreType.DMA for _ in range(3)],
    ] + _row_scratch(),
)
def _fused_pass(dst_hbm, ew_hbm, packed_hbm, xl_hbm, acc_out, coeff_out,
                prodloc, dstring, ewring, valsb, onesb, outv, zv,
                d_sh, c_sh, sem_s,
                ebuf, srcv, dstv, rows, coeffloc, acc_sh, sem_i, sem_g):
    c = lax.axis_index("c")
    s = lax.axis_index("s")

    def zbody(j, _):
        zv[pl.ds(j * 16, 16)] = jnp.zeros((16,), jnp.float32)
        return 0
    lax.fori_loop(0, STRIPE // 16, zbody, 0)
    pltpu.sync_copy(zv, d_sh.at[pl.ds(s * STRIPE, STRIPE)])
    pltpu.sync_copy(zv, c_sh.at[pl.ds(s * STRIPE, STRIPE)])
    plsc.subcore_barrier()

    ET1 = EP // NS
    NCH1 = ET1 // KS
    p1base = s * ET1

    def istart1(i, b):
        pltpu.async_copy(dst_hbm.at[pl.ds(p1base + i * KS, KS)],
                         dstring[b], sem_s[b])
        pltpu.async_copy(ew_hbm.at[pl.ds(p1base + i * KS, KS)],
                         ewring[b], sem_s[b])

    def iwait1(i, b):
        pltpu.make_async_copy(dst_hbm.at[pl.ds(p1base + i * KS, KS)],
                              dstring[b], sem_s[b]).wait()
        pltpu.make_async_copy(ew_hbm.at[pl.ds(p1base + i * KS, KS)],
                              ewring[b], sem_s[b]).wait()

    istart1(0, 0)
    istart1(1, 1)
    for i in range(NCH1):
        b = i % 3
        iwait1(i, b)

        def compute(g, _):
            sl = pl.ds(g * 16, 16)
            w16 = ewring[b][sl]
            valsb[sl] = jnp.exp(w16)
            onesb[sl] = jnp.where(w16 > jnp.float32(-50.0),
                                  jnp.float32(1.0), jnp.float32(0.0))
            return 0
        lax.fori_loop(0, KS // 16, compute, 0, unroll=2)
        pltpu.sync_copy(valsb, d_sh.at[dstring[b]], add=True)
        pltpu.sync_copy(onesb, c_sh.at[dstring[b]], add=True)
        if i + 2 < NCH1:
            istart1(i + 2, (i + 2) % 3)
    plsc.subcore_barrier()

    pltpu.sync_copy(d_sh, prodloc)
    pltpu.sync_copy(c_sh, coeffloc.at[pl.ds(0, N2)])

    def comb(j, _):
        sl = pl.ds(j * 16, 16)
        prodloc[sl] = ((prodloc[sl] + jnp.float32(EXP1))
                       * (coeffloc[sl] + jnp.float32(1.0)))
        return 0
    lax.fori_loop(0, N2 // 16, comb, 0, unroll=4)

    gbase = c * (EP2 // NC) + s * ETR
    NCH2 = ETR // KS
    iot = _iota16()

    def istart2(i, b):
        pltpu.async_copy(dst_hbm.at[pl.ds(gbase + i * KS, KS)],
                         dstring[b], sem_s[b])
        pltpu.async_copy(ew_hbm.at[pl.ds(gbase + i * KS, KS)],
                         ewring[b], sem_s[b])

    def iwait2(i, b):
        pltpu.make_async_copy(dst_hbm.at[pl.ds(gbase + i * KS, KS)],
                              dstring[b], sem_s[b]).wait()
        pltpu.make_async_copy(ew_hbm.at[pl.ds(gbase + i * KS, KS)],
                              ewring[b], sem_s[b]).wait()

    istart2(0, 0)
    istart2(1, 1)
    for i in range(NCH2):
        b = i % 3
        iwait2(i, b)

        def compute2(g, _):
            sl = pl.ds(g * 16, 16)
            e16 = iot + (gbase + i * KS + g * 16)
            dst16 = dstring[b][sl]
            w16 = ewring[b][sl]
            realc = jnp.exp(w16) / plsc.load_gather(prodloc, [dst16])
            n16 = jnp.clip(e16 - jnp.int32(EP), jnp.int32(0),
                           jnp.int32(N2 - 1))
            selfc = jnp.float32(EXP1) / plsc.load_gather(prodloc, [n16])
            cf = jnp.where(e16 < jnp.int32(EP), realc,
                           jnp.where(e16 < jnp.int32(EP + N2), selfc,
                                     jnp.float32(0.0)))
            outv[sl] = cf
            coeffloc[pl.ds(i * KS + g * 16, 16)] = cf
            return 0
        lax.fori_loop(0, KS // 16, compute2, 0, unroll=2)
        pltpu.sync_copy(outv, coeff_out.at[pl.ds(gbase + i * KS, KS)])
        if i + 2 < NCH2:
            istart2(i + 2, (i + 2) % 3)

    _row_phase(c, s, packed_hbm, xl_hbm, acc_out,
               ebuf, srcv, dstv, rows, coeffloc, acc_sh, sem_i, sem_g)


@functools.partial(
    pl.kernel,
    mesh=plsc.VectorSubcoreMesh(**_MESH),
    compiler_params=pltpu.CompilerParams(needs_layout_passes=False),
    out_type=jax.ShapeDtypeStruct((NC, N2, D), jnp.float32),
    scratch_types=_row_scratch(),
)
def _row_pass(packed_hbm, coeff_hbm, xl_hbm, acc_out,
              ebuf, srcv, dstv, rows, coeffloc, acc_sh, sem_i, sem_g):
    c = lax.axis_index("c")
    s = lax.axis_index("s")
    gbase = c * (EP2 // NC) + s * ETR
    pltpu.sync_copy(coeff_hbm.at[pl.ds(gbase, ETR)], coeffloc)
    _row_phase(c, s, packed_hbm, xl_hbm, acc_out,
               ebuf, srcv, dstv, rows, coeffloc, acc_sh, sem_i, sem_g)


def _mm_bias(xin, w, b):
    m = xin.shape[0]
    bm = 1280

    def body(x_ref, w_ref, b_ref, o_ref):
        o_ref[...] = jnp.dot(x_ref[...], w_ref[...],
                             preferred_element_type=jnp.float32) + b_ref[...]

    return pl.pallas_call(
        body,
        grid=(m // bm,),
        in_specs=[
            pl.BlockSpec((bm, D), lambda i: (i, 0)),
            pl.BlockSpec((D, D), lambda i: (0, 0)),
            pl.BlockSpec((1, D), lambda i: (0, 0)),
        ],
        out_specs=pl.BlockSpec((bm, D), lambda i: (i, 0)),
        out_shape=jax.ShapeDtypeStruct((m, D), jnp.float32),
    )(xin, w, b.reshape(1, D))


def _post(acc, w, relu):
    bm = 1280

    def body(a0_ref, a1_ref, w_ref, o_ref):
        aggr = a0_ref[0] + a1_ref[0]
        o = jnp.dot(aggr, w_ref[...], preferred_element_type=jnp.float32)
        if relu:
            o = jnp.maximum(o, jnp.float32(0.0))
        o_ref[...] = o

    return pl.pallas_call(
        body,
        grid=(N2 // bm,),
        in_specs=[
            pl.BlockSpec((1, bm, D), lambda i: (0, i, 0)),
            pl.BlockSpec((1, bm, D), lambda i: (1, i, 0)),
            pl.BlockSpec((D, D), lambda i: (0, 0)),
        ],
        out_specs=pl.BlockSpec((bm, D), lambda i: (i, 0)),
        out_shape=jax.ShapeDtypeStruct((N2, D), jnp.float32),
    )(acc, acc, w)


def kernel(x, edge_index, edge_weight, lin1_w, lin1_b, w1, lin2_w, lin2_b, w2):
    src = edge_index[0]
    dst = edge_index[1]
    pad1 = EP - E
    tail = EP2 - EP - N2
    sp1 = jnp.arange(pad1, dtype=jnp.int32) % N
    spt = jnp.arange(tail, dtype=jnp.int32) % N
    nn = jnp.arange(N2, dtype=jnp.int32)
    nsrc = nn % N

    dst2 = jnp.concatenate([dst, sp1, nn, spt])
    ew2 = jnp.concatenate([edge_weight,
                           jnp.full((EP2 - E,), -100.0, jnp.float32)])
    packed = jnp.concatenate([
        jnp.bitwise_or(src, jnp.left_shift(dst, 14)),
        jnp.bitwise_or(sp1, jnp.left_shift(sp1, 14)),
        jnp.bitwise_or(nsrc, jnp.left_shift(nn, 14)),
        jnp.bitwise_or(spt, jnp.left_shift(spt, 14)),
    ])

    x2 = jnp.concatenate([x, jnp.zeros((N2 - N, D), jnp.float32)])
    xl1 = _mm_bias(x2, lin1_w, lin1_b)
    acc1, coeff = _fused_pass(dst2, ew2, packed, xl1)
    h = _post(acc1, w1, relu=True)

    xl2 = _mm_bias(h, lin2_w, lin2_b)
    acc2 = _row_pass(packed, coeff, xl2)
    out = _post(acc2, w2, relu=False)
    return out[:N]
